# initial kernel scaffold (unmeasured)
import jax
import jax.numpy as jnp
from jax import lax
from jax.experimental import pallas as pl
from jax.experimental.pallas import tpu as pltpu

N_DEV = 8


def _layer(x, Win, Wout, collective_id):
    B, Kin = x.shape
    H = Win.shape[1]
    Nout = Wout.shape[1]
    Hc = H // N_DEV

    def body(x_ref, win_ref, wout_ref, out_ref,
             acc_ref, hfull_ref, comm_ref,
             rs_send_sems, rs_recv_sems, ag_send_sems, ag_recv_sems):
        my = lax.axis_index("i")
        left = lax.rem(my + N_DEV - 1, N_DEV)
        right = lax.rem(my + 1, N_DEV)

        barrier_sem = pltpu.get_barrier_semaphore()
        for nbr in (left, right):
            pl.semaphore_signal(
                barrier_sem, inc=1,
                device_id=(nbr,), device_id_type=pl.DeviceIdType.MESH,
            )
        pl.semaphore_wait(barrier_sem, 2)

        acc_ref[...] = jnp.dot(
            x_ref[...], win_ref[...], preferred_element_type=jnp.float32
        )

        for s in range(N_DEV - 1):
            send_c = lax.rem(my + N_DEV - 1 - s, N_DEV)
            recv_c = lax.rem(my + N_DEV - 2 - s, N_DEV)
            rdma = pltpu.make_async_remote_copy(
                src_ref=acc_ref.at[:, pl.ds(send_c * Hc, Hc)],
                dst_ref=comm_ref.at[s],
                send_sem=rs_send_sems.at[s],
                recv_sem=rs_recv_sems.at[s],
                device_id=(right,),
                device_id_type=pl.DeviceIdType.MESH,
            )
            rdma.start()
            rdma.wait()
            sl = pl.ds(recv_c * Hc, Hc)
            acc_ref[:, sl] = acc_ref[:, sl] + comm_ref[s]

        sl_my = pl.ds(my * Hc, Hc)
        hfull_ref[:, sl_my] = jnp.maximum(acc_ref[:, sl_my], 0.0)

        for s in range(N_DEV - 1):
            send_c = lax.rem(my + N_DEV - s, N_DEV)
            rdma = pltpu.make_async_remote_copy(
                src_ref=hfull_ref.at[:, pl.ds(send_c * Hc, Hc)],
                dst_ref=hfull_ref.at[:, pl.ds(send_c * Hc, Hc)],
                send_sem=ag_send_sems.at[s],
                recv_sem=ag_recv_sems.at[s],
                device_id=(right,),
                device_id_type=pl.DeviceIdType.MESH,
            )
            rdma.start()
            rdma.wait()

        out_ref[...] = jnp.dot(
            hfull_ref[...], wout_ref[...], preferred_element_type=jnp.float32
        )

    return pl.pallas_call(
        body,
        out_shape=jax.ShapeDtypeStruct((B, Nout), jnp.float32),
        in_specs=[
            pl.BlockSpec(memory_space=pltpu.VMEM),
            pl.BlockSpec(memory_space=pltpu.VMEM),
            pl.BlockSpec(memory_space=pltpu.VMEM),
        ],
        out_specs=pl.BlockSpec(memory_space=pltpu.VMEM),
        scratch_shapes=[
            pltpu.VMEM((B, H), jnp.float32),
            pltpu.VMEM((B, H), jnp.float32),
            pltpu.VMEM((N_DEV - 1, B, Hc), jnp.float32),
            pltpu.SemaphoreType.DMA((N_DEV - 1,)),
            pltpu.SemaphoreType.DMA((N_DEV - 1,)),
            pltpu.SemaphoreType.DMA((N_DEV - 1,)),
            pltpu.SemaphoreType.DMA((N_DEV - 1,)),
        ],
        compiler_params=pltpu.CompilerParams(collective_id=collective_id),
    )(x, Win, Wout)


def kernel(x, Win0, Wout0, Win1, Wout1, Win2, Wout2):
    x = _layer(x, Win0, Wout0, 0)
    x = _layer(x, Win1, Wout1, 1)
    x = _layer(x, Win2, Wout2, 2)
    return x


# baseline (device time: 230835 ns/iter reference)
import jax
import jax.numpy as jnp
from jax import lax
from jax.experimental import pallas as pl
from jax.experimental.pallas import tpu as pltpu

N_DEV = 8


def _hidden_allreduce(x, Win, collective_id):
    B, Kin = x.shape
    H = Win.shape[1]
    Hc = H // N_DEV

    def body(x_ref, win_ref, hfull_ref,
             acc_ref, comm_ref,
             rs_send_sems, rs_recv_sems, ag_send_sems, ag_recv_sems):
        my = lax.axis_index("i")
        left = lax.rem(my + N_DEV - 1, N_DEV)
        right = lax.rem(my + 1, N_DEV)

        barrier_sem = pltpu.get_barrier_semaphore()
        for nbr in (left, right):
            pl.semaphore_signal(
                barrier_sem, inc=1,
                device_id=(nbr,), device_id_type=pl.DeviceIdType.MESH,
            )
        pl.semaphore_wait(barrier_sem, 2)

        acc_ref[...] = jnp.dot(
            x_ref[...], win_ref[...], preferred_element_type=jnp.float32
        )

        for s in range(N_DEV - 1):
            send_c = lax.rem(my + N_DEV - 1 - s, N_DEV)
            recv_c = lax.rem(my + N_DEV - 2 - s, N_DEV)
            rdma = pltpu.make_async_remote_copy(
                src_ref=acc_ref.at[:, pl.ds(send_c * Hc, Hc)],
                dst_ref=comm_ref.at[s],
                send_sem=rs_send_sems.at[s],
                recv_sem=rs_recv_sems.at[s],
                device_id=(right,),
                device_id_type=pl.DeviceIdType.MESH,
            )
            rdma.start()
            rdma.wait()
            sl = pl.ds(recv_c * Hc, Hc)
            acc_ref[:, sl] = acc_ref[:, sl] + comm_ref[s]

        sl_my = pl.ds(my * Hc, Hc)
        hfull_ref[:, sl_my] = jnp.maximum(acc_ref[:, sl_my], 0.0)

        for s in range(N_DEV - 1):
            send_c = lax.rem(my + N_DEV - s, N_DEV)
            rdma = pltpu.make_async_remote_copy(
                src_ref=hfull_ref.at[:, pl.ds(send_c * Hc, Hc)],
                dst_ref=hfull_ref.at[:, pl.ds(send_c * Hc, Hc)],
                send_sem=ag_send_sems.at[s],
                recv_sem=ag_recv_sems.at[s],
                device_id=(right,),
                device_id_type=pl.DeviceIdType.MESH,
            )
            rdma.start()
            rdma.wait()

    return pl.pallas_call(
        body,
        out_shape=jax.ShapeDtypeStruct((B, H), jnp.float32),
        in_specs=[
            pl.BlockSpec(memory_space=pltpu.VMEM),
            pl.BlockSpec(memory_space=pltpu.VMEM),
        ],
        out_specs=pl.BlockSpec(memory_space=pltpu.VMEM),
        scratch_shapes=[
            pltpu.VMEM((B, H), jnp.float32),
            pltpu.VMEM((N_DEV - 1, B, Hc), jnp.float32),
            pltpu.SemaphoreType.DMA((N_DEV - 1,)),
            pltpu.SemaphoreType.DMA((N_DEV - 1,)),
            pltpu.SemaphoreType.DMA((N_DEV - 1,)),
            pltpu.SemaphoreType.DMA((N_DEV - 1,)),
        ],
        compiler_params=pltpu.CompilerParams(
            collective_id=collective_id, vmem_limit_bytes=60 * 1024 * 1024
        ),
    )(x, Win)


def _out_matmul(h, Wout):
    B = h.shape[0]
    Nout = Wout.shape[1]

    def body(h_ref, wout_ref, out_ref):
        out_ref[...] = jnp.dot(
            h_ref[...], wout_ref[...], preferred_element_type=jnp.float32
        )

    return pl.pallas_call(
        body,
        out_shape=jax.ShapeDtypeStruct((B, Nout), jnp.float32),
        in_specs=[
            pl.BlockSpec(memory_space=pltpu.VMEM),
            pl.BlockSpec(memory_space=pltpu.VMEM),
        ],
        out_specs=pl.BlockSpec(memory_space=pltpu.VMEM),
        compiler_params=pltpu.CompilerParams(
            vmem_limit_bytes=60 * 1024 * 1024
        ),
    )(h, Wout)


def kernel(x, Win0, Wout0, Win1, Wout1, Win2, Wout2):
    for cid, (Win, Wout) in enumerate(((Win0, Wout0), (Win1, Wout1), (Win2, Wout2))):
        h = _hidden_allreduce(x, Win, cid)
        x = _out_matmul(h, Wout)
    return x


# device time: 141452 ns/iter; 1.6319x vs baseline; 1.6319x over previous
import jax
import jax.numpy as jnp
from jax import lax
from jax.experimental import pallas as pl
from jax.experimental.pallas import tpu as pltpu

N_DEV = 8
VMEM_LIMIT = 60 * 1024 * 1024


def _hidden_allreduce(x, Win, collective_id):
    B, Kin = x.shape
    H = Win.shape[1]
    Hc = H // N_DEV

    def body(x_ref, win_ref, hfull_ref,
             acc_ref, comm_ref,
             rs_send_sems, rs_recv_sems, ag_send_sems, ag_recv_sems):
        my = lax.axis_index("i")

        barrier_sem = pltpu.get_barrier_semaphore()
        for k in range(1, N_DEV):
            p = lax.rem(my + k, N_DEV)
            pl.semaphore_signal(
                barrier_sem, inc=1,
                device_id=(p,), device_id_type=pl.DeviceIdType.MESH,
            )
        pl.semaphore_wait(barrier_sem, N_DEV - 1)

        own_blk = pl.ds(my * Hc, Hc)

        sends = []
        for k in range(1, N_DEV):
            p = lax.rem(my + k, N_DEV)
            blk = pl.ds(p * Hc, Hc)
            acc_ref[:, blk] = jnp.dot(
                x_ref[...], win_ref[:, blk],
                preferred_element_type=jnp.float32,
            )
            rdma = pltpu.make_async_remote_copy(
                src_ref=acc_ref.at[:, blk],
                dst_ref=comm_ref.at[my],
                send_sem=rs_send_sems.at[p],
                recv_sem=rs_recv_sems.at[my],
                device_id=(p,),
                device_id_type=pl.DeviceIdType.MESH,
            )
            rdma.start()
            sends.append(rdma)

        acc_ref[:, own_blk] = jnp.dot(
            x_ref[...], win_ref[:, own_blk],
            preferred_element_type=jnp.float32,
        )

        red = acc_ref[:, own_blk]
        for k in range(1, N_DEV):
            s = lax.rem(my + N_DEV - k, N_DEV)
            recv = pltpu.make_async_remote_copy(
                src_ref=acc_ref.at[:, own_blk],
                dst_ref=comm_ref.at[s],
                send_sem=rs_send_sems.at[s],
                recv_sem=rs_recv_sems.at[s],
                device_id=(my,),
                device_id_type=pl.DeviceIdType.MESH,
            )
            recv.wait_recv()
            red = red + comm_ref[s]

        hfull_ref[:, own_blk] = jnp.maximum(red, 0.0)

        ag_sends = []
        for k in range(1, N_DEV):
            p = lax.rem(my + k, N_DEV)
            rdma = pltpu.make_async_remote_copy(
                src_ref=hfull_ref.at[:, own_blk],
                dst_ref=hfull_ref.at[:, own_blk],
                send_sem=ag_send_sems.at[p],
                recv_sem=ag_recv_sems.at[my],
                device_id=(p,),
                device_id_type=pl.DeviceIdType.MESH,
            )
            rdma.start()
            ag_sends.append(rdma)

        for k in range(1, N_DEV):
            s = lax.rem(my + N_DEV - k, N_DEV)
            recv = pltpu.make_async_remote_copy(
                src_ref=hfull_ref.at[:, own_blk],
                dst_ref=hfull_ref.at[:, pl.ds(s * Hc, Hc)],
                send_sem=ag_send_sems.at[s],
                recv_sem=ag_recv_sems.at[s],
                device_id=(my,),
                device_id_type=pl.DeviceIdType.MESH,
            )
            recv.wait_recv()

        for rdma in sends + ag_sends:
            rdma.wait_send()

    return pl.pallas_call(
        body,
        out_shape=jax.ShapeDtypeStruct((B, H), jnp.float32),
        in_specs=[
            pl.BlockSpec(memory_space=pltpu.VMEM),
            pl.BlockSpec(memory_space=pltpu.VMEM),
        ],
        out_specs=pl.BlockSpec(memory_space=pltpu.VMEM),
        scratch_shapes=[
            pltpu.VMEM((B, H), jnp.float32),
            pltpu.VMEM((N_DEV, B, Hc), jnp.float32),
            pltpu.SemaphoreType.DMA((N_DEV,)),
            pltpu.SemaphoreType.DMA((N_DEV,)),
            pltpu.SemaphoreType.DMA((N_DEV,)),
            pltpu.SemaphoreType.DMA((N_DEV,)),
        ],
        compiler_params=pltpu.CompilerParams(
            collective_id=collective_id, vmem_limit_bytes=VMEM_LIMIT
        ),
    )(x, Win)


def _out_matmul(h, Wout):
    B = h.shape[0]
    Nout = Wout.shape[1]

    def body(h_ref, wout_ref, out_ref):
        out_ref[...] = jnp.dot(
            h_ref[...], wout_ref[...], preferred_element_type=jnp.float32
        )

    return pl.pallas_call(
        body,
        out_shape=jax.ShapeDtypeStruct((B, Nout), jnp.float32),
        in_specs=[
            pl.BlockSpec(memory_space=pltpu.VMEM),
            pl.BlockSpec(memory_space=pltpu.VMEM),
        ],
        out_specs=pl.BlockSpec(memory_space=pltpu.VMEM),
        compiler_params=pltpu.CompilerParams(vmem_limit_bytes=VMEM_LIMIT),
    )(h, Wout)


def kernel(x, Win0, Wout0, Win1, Wout1, Win2, Wout2):
    for cid, (Win, Wout) in enumerate(((Win0, Wout0), (Win1, Wout1), (Win2, Wout2))):
        h = _hidden_allreduce(x, Win, cid)
        x = _out_matmul(h, Wout)
    return x


# device time: 111742 ns/iter; 2.0658x vs baseline; 1.2659x over previous
import jax
import jax.numpy as jnp
from jax import lax
from jax.experimental import pallas as pl
from jax.experimental.pallas import tpu as pltpu

N_DEV = 8
VMEM_LIMIT = 60 * 1024 * 1024


def kernel(x, Win0, Wout0, Win1, Wout1, Win2, Wout2):
    B, Kin = x.shape
    H = Win0.shape[1]
    Nout = Wout0.shape[1]
    Hc = H // N_DEV

    def body(x_ref, win0_ref, wout0_ref, win1_ref, wout1_ref,
             win2_ref, wout2_ref, out_ref,
             xbuf_ref, acc_ref, hfull_ref, comm_ref,
             win_buf, wout_buf,
             win_sems, wout_sems,
             rs_send_sems, rs_recv_sems, ag_send_sems, ag_recv_sems):
        my = lax.axis_index("i")

        barrier_sem = pltpu.get_barrier_semaphore()
        for k in range(1, N_DEV):
            p = lax.rem(my + k, N_DEV)
            pl.semaphore_signal(
                barrier_sem, inc=1,
                device_id=(p,), device_id_type=pl.DeviceIdType.MESH,
            )
        pl.semaphore_wait(barrier_sem, N_DEV - 1)

        own_blk = pl.ds(my * Hc, Hc)

        layers = (
            (x_ref, win0_ref, wout0_ref, xbuf_ref.at[0]),
            (xbuf_ref.at[0], win1_ref, wout1_ref, xbuf_ref.at[1]),
            (xbuf_ref.at[1], win2_ref, wout2_ref, out_ref),
        )

        for xin_ref, win_hbm, wout_hbm, xout_ref in layers:
            for sblk in range(N_DEV):
                pltpu.make_async_copy(
                    wout_hbm.at[pl.ds(sblk * Hc, Hc), :],
                    wout_buf.at[sblk],
                    wout_sems.at[sblk],
                ).start()

            def win_tile_copy(t, buf):
                p = lax.rem(my + 1 + t, N_DEV) if t < N_DEV - 1 else my
                return pltpu.make_async_copy(
                    win_hbm.at[:, pl.ds(p * Hc, Hc)],
                    win_buf.at[buf],
                    win_sems.at[buf],
                )

            win_tile_copy(0, 0).start()
            rs_sends = []
            for t in range(N_DEV):
                buf = t % 2
                if t + 1 < N_DEV:
                    win_tile_copy(t + 1, (t + 1) % 2).start()
                if t < N_DEV - 1:
                    p = lax.rem(my + 1 + t, N_DEV)
                else:
                    p = my
                blk = pl.ds(p * Hc, Hc)
                win_tile_copy(t, buf).wait()
                acc_ref[:, blk] = jnp.dot(
                    xin_ref[...], win_buf[buf],
                    preferred_element_type=jnp.float32,
                )
                if t < N_DEV - 1:
                    rdma = pltpu.make_async_remote_copy(
                        src_ref=acc_ref.at[:, blk],
                        dst_ref=comm_ref.at[my],
                        send_sem=rs_send_sems.at[p],
                        recv_sem=rs_recv_sems.at[my],
                        device_id=(p,),
                        device_id_type=pl.DeviceIdType.MESH,
                    )
                    rdma.start()
                    rs_sends.append(rdma)

            red = acc_ref[:, own_blk]
            for k in range(1, N_DEV):
                s = lax.rem(my + N_DEV - k, N_DEV)
                recv = pltpu.make_async_remote_copy(
                    src_ref=acc_ref.at[:, own_blk],
                    dst_ref=comm_ref.at[s],
                    send_sem=rs_send_sems.at[s],
                    recv_sem=rs_recv_sems.at[s],
                    device_id=(my,),
                    device_id_type=pl.DeviceIdType.MESH,
                )
                recv.wait_recv()
                red = red + comm_ref[s]
            hfull_ref[:, own_blk] = jnp.maximum(red, 0.0)

            ag_sends = []
            for k in range(1, N_DEV):
                p = lax.rem(my + k, N_DEV)
                rdma = pltpu.make_async_remote_copy(
                    src_ref=hfull_ref.at[:, own_blk],
                    dst_ref=hfull_ref.at[:, own_blk],
                    send_sem=ag_send_sems.at[p],
                    recv_sem=ag_recv_sems.at[my],
                    device_id=(p,),
                    device_id_type=pl.DeviceIdType.MESH,
                )
                rdma.start()
                ag_sends.append(rdma)

            def wait_wout(sblk_dyn):
                pltpu.make_async_copy(
                    wout_hbm.at[pl.ds(sblk_dyn * Hc, Hc), :],
                    wout_buf.at[sblk_dyn],
                    wout_sems.at[sblk_dyn],
                ).wait()

            wait_wout(my)
            xout_ref[...] = jnp.dot(
                hfull_ref[:, own_blk], wout_buf[my],
                preferred_element_type=jnp.float32,
            )
            for k in range(1, N_DEV):
                s = lax.rem(my + N_DEV - k, N_DEV)
                recv = pltpu.make_async_remote_copy(
                    src_ref=hfull_ref.at[:, own_blk],
                    dst_ref=hfull_ref.at[:, pl.ds(s * Hc, Hc)],
                    send_sem=ag_send_sems.at[s],
                    recv_sem=ag_recv_sems.at[s],
                    device_id=(my,),
                    device_id_type=pl.DeviceIdType.MESH,
                )
                recv.wait_recv()
                wait_wout(s)
                xout_ref[...] = xout_ref[...] + jnp.dot(
                    hfull_ref[:, pl.ds(s * Hc, Hc)], wout_buf[s],
                    preferred_element_type=jnp.float32,
                )

            for rdma in rs_sends + ag_sends:
                rdma.wait_send()

    return pl.pallas_call(
        body,
        out_shape=jax.ShapeDtypeStruct((B, Nout), jnp.float32),
        in_specs=[
            pl.BlockSpec(memory_space=pltpu.VMEM),
            pl.BlockSpec(memory_space=pltpu.MemorySpace.HBM),
            pl.BlockSpec(memory_space=pltpu.MemorySpace.HBM),
            pl.BlockSpec(memory_space=pltpu.MemorySpace.HBM),
            pl.BlockSpec(memory_space=pltpu.MemorySpace.HBM),
            pl.BlockSpec(memory_space=pltpu.MemorySpace.HBM),
            pl.BlockSpec(memory_space=pltpu.MemorySpace.HBM),
        ],
        out_specs=pl.BlockSpec(memory_space=pltpu.VMEM),
        scratch_shapes=[
            pltpu.VMEM((2, B, Kin), jnp.float32),
            pltpu.VMEM((B, H), jnp.float32),
            pltpu.VMEM((B, H), jnp.float32),
            pltpu.VMEM((N_DEV, B, Hc), jnp.float32),
            pltpu.VMEM((2, Kin, Hc), jnp.float32),
            pltpu.VMEM((N_DEV, Hc, Nout), jnp.float32),
            pltpu.SemaphoreType.DMA((2,)),
            pltpu.SemaphoreType.DMA((N_DEV,)),
            pltpu.SemaphoreType.DMA((N_DEV,)),
            pltpu.SemaphoreType.DMA((N_DEV,)),
            pltpu.SemaphoreType.DMA((N_DEV,)),
            pltpu.SemaphoreType.DMA((N_DEV,)),
        ],
        compiler_params=pltpu.CompilerParams(
            collective_id=0, vmem_limit_bytes=VMEM_LIMIT
        ),
    )(x, Win0, Wout0, Win1, Wout1, Win2, Wout2)


# device time: 98495 ns/iter; 2.3436x vs baseline; 1.1345x over previous
import jax
import jax.numpy as jnp
from jax import lax
from jax.experimental import pallas as pl
from jax.experimental.pallas import tpu as pltpu

N_DEV = 8
VMEM_LIMIT = 60 * 1024 * 1024


def kernel(x, Win0, Wout0, Win1, Wout1, Win2, Wout2):
    B, Kin = x.shape
    H = Win0.shape[1]
    Nout = Wout0.shape[1]
    Hc = H // N_DEV

    def body(x_ref, win0_ref, wout0_ref, win1_ref, wout1_ref,
             win2_ref, wout2_ref, out_ref,
             xbuf_ref, acc_ref, hfull_ref, comm_ref,
             win_buf, wout_buf,
             win_sems, wout_sems,
             rs_send_sems, rs_recv_sems, ag_send_sems, ag_recv_sems):
        my = lax.axis_index("i")

        barrier_sem = pltpu.get_barrier_semaphore()
        for k in range(1, N_DEV):
            p = lax.rem(my + k, N_DEV)
            pl.semaphore_signal(
                barrier_sem, inc=1,
                device_id=(p,), device_id_type=pl.DeviceIdType.MESH,
            )
        pl.semaphore_wait(barrier_sem, N_DEV - 1)

        own_blk = pl.ds(my * Hc, Hc)

        layers = (
            (x_ref, win0_ref, wout0_ref, xbuf_ref.at[0]),
            (xbuf_ref.at[0], win1_ref, wout1_ref, xbuf_ref.at[1]),
            (xbuf_ref.at[1], win2_ref, wout2_ref, out_ref),
        )

        def win_tile_copy(win_hbm, t, buf):
            p = lax.rem(my + 1 + t, N_DEV) if t < N_DEV - 1 else my
            return pltpu.make_async_copy(
                win_hbm.at[:, pl.ds(p * Hc, Hc)],
                win_buf.at[buf],
                win_sems.at[buf],
            )

        win_tile_copy(layers[0][1], 0, 0).start()

        for li, (xin_ref, win_hbm, wout_hbm, xout_ref) in enumerate(layers):
            rs_sends = []
            for t in range(N_DEV):
                buf = t % 2
                if t + 1 < N_DEV:
                    win_tile_copy(win_hbm, t + 1, (t + 1) % 2).start()
                pltpu.make_async_copy(
                    wout_hbm.at[pl.ds(t * Hc, Hc), :],
                    wout_buf.at[t],
                    wout_sems.at[t],
                ).start()
                if t < N_DEV - 1:
                    p = lax.rem(my + 1 + t, N_DEV)
                else:
                    p = my
                blk = pl.ds(p * Hc, Hc)
                win_tile_copy(win_hbm, t, buf).wait()
                acc_ref[:, blk] = jnp.dot(
                    xin_ref[...], win_buf[buf],
                    preferred_element_type=jnp.float32,
                )
                if t < N_DEV - 1:
                    rdma = pltpu.make_async_remote_copy(
                        src_ref=acc_ref.at[:, blk],
                        dst_ref=comm_ref.at[my],
                        send_sem=rs_send_sems.at[p],
                        recv_sem=rs_recv_sems.at[my],
                        device_id=(p,),
                        device_id_type=pl.DeviceIdType.MESH,
                    )
                    rdma.start()
                    rs_sends.append(rdma)

            red = acc_ref[:, own_blk]
            for k in range(1, N_DEV):
                s = lax.rem(my + N_DEV - k, N_DEV)
                recv = pltpu.make_async_remote_copy(
                    src_ref=acc_ref.at[:, own_blk],
                    dst_ref=comm_ref.at[s],
                    send_sem=rs_send_sems.at[s],
                    recv_sem=rs_recv_sems.at[s],
                    device_id=(my,),
                    device_id_type=pl.DeviceIdType.MESH,
                )
                recv.wait_recv()
                red = red + comm_ref[s]
            hfull_ref[:, own_blk] = jnp.maximum(red, 0.0)

            ag_sends = []
            for k in range(1, N_DEV):
                p = lax.rem(my + k, N_DEV)
                rdma = pltpu.make_async_remote_copy(
                    src_ref=hfull_ref.at[:, own_blk],
                    dst_ref=hfull_ref.at[:, own_blk],
                    send_sem=ag_send_sems.at[p],
                    recv_sem=ag_recv_sems.at[my],
                    device_id=(p,),
                    device_id_type=pl.DeviceIdType.MESH,
                )
                rdma.start()
                ag_sends.append(rdma)

            if li + 1 < len(layers):
                win_tile_copy(layers[li + 1][1], 0, 0).start()

            def wait_wout(sblk_dyn):
                pltpu.make_async_copy(
                    wout_hbm.at[pl.ds(sblk_dyn * Hc, Hc), :],
                    wout_buf.at[sblk_dyn],
                    wout_sems.at[sblk_dyn],
                ).wait()

            wait_wout(my)
            xout_ref[...] = jnp.dot(
                hfull_ref[:, own_blk], wout_buf[my],
                preferred_element_type=jnp.float32,
            )
            for k in range(1, N_DEV):
                s = lax.rem(my + N_DEV - k, N_DEV)
                recv = pltpu.make_async_remote_copy(
                    src_ref=hfull_ref.at[:, own_blk],
                    dst_ref=hfull_ref.at[:, pl.ds(s * Hc, Hc)],
                    send_sem=ag_send_sems.at[s],
                    recv_sem=ag_recv_sems.at[s],
                    device_id=(my,),
                    device_id_type=pl.DeviceIdType.MESH,
                )
                recv.wait_recv()
                wait_wout(s)
                xout_ref[...] = xout_ref[...] + jnp.dot(
                    hfull_ref[:, pl.ds(s * Hc, Hc)], wout_buf[s],
                    preferred_element_type=jnp.float32,
                )

            for rdma in rs_sends + ag_sends:
                rdma.wait_send()

    return pl.pallas_call(
        body,
        out_shape=jax.ShapeDtypeStruct((B, Nout), jnp.float32),
        in_specs=[
            pl.BlockSpec(memory_space=pltpu.VMEM),
            pl.BlockSpec(memory_space=pltpu.MemorySpace.HBM),
            pl.BlockSpec(memory_space=pltpu.MemorySpace.HBM),
            pl.BlockSpec(memory_space=pltpu.MemorySpace.HBM),
            pl.BlockSpec(memory_space=pltpu.MemorySpace.HBM),
            pl.BlockSpec(memory_space=pltpu.MemorySpace.HBM),
            pl.BlockSpec(memory_space=pltpu.MemorySpace.HBM),
        ],
        out_specs=pl.BlockSpec(memory_space=pltpu.VMEM),
        scratch_shapes=[
            pltpu.VMEM((2, B, Kin), jnp.float32),
            pltpu.VMEM((B, H), jnp.float32),
            pltpu.VMEM((B, H), jnp.float32),
            pltpu.VMEM((N_DEV, B, Hc), jnp.float32),
            pltpu.VMEM((2, Kin, Hc), jnp.float32),
            pltpu.VMEM((N_DEV, Hc, Nout), jnp.float32),
            pltpu.SemaphoreType.DMA((2,)),
            pltpu.SemaphoreType.DMA((N_DEV,)),
            pltpu.SemaphoreType.DMA((N_DEV,)),
            pltpu.SemaphoreType.DMA((N_DEV,)),
            pltpu.SemaphoreType.DMA((N_DEV,)),
            pltpu.SemaphoreType.DMA((N_DEV,)),
        ],
        compiler_params=pltpu.CompilerParams(
            collective_id=0, vmem_limit_bytes=VMEM_LIMIT
        ),
    )(x, Win0, Wout0, Win1, Wout1, Win2, Wout2)
